# baseline (speedup 1.0000x reference)
import jax
import jax.numpy as jnp
import numpy as np
from jax.experimental import pallas as pl
from jax.experimental.pallas import tpu as pltpu

_PAD = 3
_MR = 11
_MC = 21
_TH = 32
_C = 16
_O = 64
_W = 192
_NSEG = 49
_KDIM = 800


def _tables():
    basis = []
    bidx = {}

    def bid(ar, lr, ac, lc):
        key = (ar, lr, ac, lc)
        if key not in bidx:
            bidx[key] = len(basis)
            basis.append(key)
        return bidx[key]

    T = np.zeros((_NSEG, _NSEG), np.float32)
    p = 0
    for i in (-1, 0, 1):
        for j in (-1, 0, 1):
            T[p, bid(i, 1, j, 1)] = 1.0
            p += 1
    for i in range(-2, 3):
        for j in range(-2, 3):
            if abs(i) != 2 and abs(j) != 2:
                continue
            if abs(i) == 2 and abs(j) <= 1:
                T[p, bid(-2, 5, j, 1)] = 1 / 5
            elif abs(j) == 2 and abs(i) <= 1:
                T[p, bid(i, 1, -2, 5)] = 1 / 5
            else:
                T[p, bid(-2, 5, j, 1)] += 1 / 9
                T[p, bid(i, 1, -1 if j == -2 else -2, 4)] += 1 / 9
            p += 1
    for i in range(-3, 4):
        for j in range(-3, 4):
            if abs(i) != 3 and abs(j) != 3:
                continue
            if abs(i) == 3 and abs(j) <= 2:
                T[p, bid(-3 if i == -3 else -11, 15, j, 1)] = 1 / 15
            elif abs(j) == 3 and abs(i) <= 2:
                T[p, bid(i, 1, -3 if j == -3 else -21, 25)] = 1 / 25
            else:
                T[p, bid(-3 if i == -3 else -11, 15, j, 1)] += 1 / 39
                T[p, bid(i, 1, -2 if j == -3 else -21, 24)] += 1 / 39
            p += 1
    assert p == _NSEG and len(basis) == _NSEG, (p, len(basis))
    return basis, T


_BASIS, _T = _tables()
_HCOMBOS = ((5, -2), (4, -1), (4, -2), (25, -3), (25, -21), (24, -2), (24, -21),
            (1, -3), (1, -2), (1, -1), (1, 0), (1, 1), (1, 2), (1, 3))
_BANDIDX = {c: i for i, c in enumerate(_HCOMBOS)}


def _hbands():
    b = np.zeros((len(_HCOMBOS), 2 * _MC + _W, _W), np.float32)
    for ci, (L, ac) in enumerate(_HCOMBOS):
        for v in range(_W):
            u0 = v + _MC + ac
            b[ci, u0:u0 + L, v] = 1.0
    return jnp.asarray(b, dtype=jnp.bfloat16)


def _fov_kernel(x_ref, w_ref, hband_ref, o_ref, f_ref, p_ref):
    t = pl.program_id(1)

    @pl.when(t == 0)
    def _build_padded():
        x = x_ref[0]
        left = jnp.concatenate(
            [jnp.broadcast_to(x[:, :, 3:4], (x.shape[0], _C, _MC - 2)),
             x[:, :, 2:3], x[:, :, 1:2]], axis=2)
        right = jnp.concatenate(
            [x[:, :, -2:-1], x[:, :, -3:-2],
             jnp.broadcast_to(x[:, :, -4:-3], (x.shape[0], _C, _MC - 2))],
            axis=2)
        body = jnp.concatenate([left, x, right], axis=2)
        p_ref[_MR:_MR + 192] = body
        p_ref[0:_MR - 2] = jnp.broadcast_to(body[3:4], (_MR - 2, _C, 234))
        p_ref[_MR - 2] = body[2]
        p_ref[_MR - 1] = body[1]
        p_ref[_MR + 192] = body[190]
        p_ref[_MR + 193] = body[189]
        p_ref[_MR + 194:] = jnp.broadcast_to(body[188:189], (_MR - 2, _C, 234))

    tile = p_ref[pl.ds(t * _TH, _TH + 2 * _MR)]

    s2 = tile[:-1] + tile[1:]
    s4 = s2[:-2] + s2[2:]
    s5 = s4[:-1] + tile[4:]
    s8 = s4[:-4] + s4[4:]
    n15 = tile.shape[0] - 14
    s15 = s8[:n15] + s4[8:8 + n15] + s2[12:12 + n15] + tile[14:]

    nhr = _TH + 6
    hb = tile[_MR - 3:_MR + 3 + _TH].astype(jnp.bfloat16)
    hb2 = hb.reshape(nhr * _C, 2 * _MC + _W)
    hres = {}
    for ci, (L, ac) in enumerate(_HCOMBOS):
        r = jnp.dot(hb2, hband_ref[ci], preferred_element_type=jnp.float32)
        hres[(L, ac)] = r.reshape(nhr, _C, _W)

    def shift_dot(arr, nr, ac):
        a2 = arr.astype(jnp.bfloat16).reshape(nr * _C, 2 * _MC + _W)
        r = jnp.dot(a2, hband_ref[_BANDIDX[(1, ac)]],
                    preferred_element_type=jnp.float32)
        return r.reshape(nr, _C, _W)

    vres = {}
    for ac in (-1, 0, 1):
        vres[('p', ac)] = shift_dot(tile[_MR - 1:_MR + 1 + _TH], _TH + 2, ac)
    for ac in (-2, -1, 0, 1, 2):
        vres[('s5', ac)] = shift_dot(s5[9:9 + _TH], _TH, ac)
    for ac in (-3, -2, -1, 0, 1, 2, 3):
        vres[('s15', ac)] = shift_dot(s15[0:_TH + 8], _TH + 8, ac)

    for m, (ar, lr, ac, lc) in enumerate(_BASIS):
        if lc == 1:
            if lr == 1:
                seg = vres[('p', ac)][ar + 1:ar + 1 + _TH]
            elif lr == 5:
                seg = vres[('s5', ac)][0:_TH]
            else:
                seg = vres[('s15', ac)][ar + 11:ar + 11 + _TH]
        else:
            seg = hres[(lc, ac)][3 + ar:3 + ar + _TH]
        f_ref[:, m * _C:(m + 1) * _C, :] = seg.astype(jnp.bfloat16)

    iota = jax.lax.broadcasted_iota(jnp.int32, (_TH, _KDIM - _NSEG * _C, _W), 1)
    f_ref[:, _NSEG * _C:, :] = jnp.where(
        iota == 0, 1.0, 0.0).astype(jnp.bfloat16)

    w = w_ref[...]
    res = [jnp.dot(w, f_ref[h], preferred_element_type=jnp.float32)
           for h in range(_TH)]
    o_ref[0] = jnp.stack(res, axis=1)


def _fold_weights(weight, bias):
    w2 = jnp.einsum('ocp,pm->omc', weight.reshape(_O, _C, _NSEG),
                    _T).reshape(_O, _NSEG * _C)
    pad = jnp.zeros((_O, _KDIM - _NSEG * _C - 1), weight.dtype)
    return jnp.concatenate([w2, bias[:, None], pad],
                           axis=1).astype(jnp.bfloat16)


def _build_call(B, H, W, interpret=False):
    nt = H // _TH
    return pl.pallas_call(
        _fov_kernel,
        out_shape=jax.ShapeDtypeStruct((B, _O, H, W), jnp.float32),
        grid=(B, nt),
        in_specs=[
            pl.BlockSpec((1, H, _C, W), lambda b, t: (b, 0, 0, 0)),
            pl.BlockSpec((_O, _KDIM), lambda b, t: (0, 0)),
            pl.BlockSpec((len(_HCOMBOS), 2 * _MC + _W, _W),
                         lambda b, t: (0, 0, 0)),
        ],
        out_specs=pl.BlockSpec((1, _O, _TH, W), lambda b, t: (b, 0, t, 0)),
        scratch_shapes=[
            pltpu.VMEM((_TH, _KDIM, _W), jnp.bfloat16),
            pltpu.VMEM((H + 2 * _MR, _C, W + 2 * _MC), jnp.float32),
        ],
        compiler_params=pltpu.CompilerParams(
            dimension_semantics=("parallel", "arbitrary"),
        ),
        name="foveated_conv",
        interpret=interpret,
    )


@jax.jit
def kernel(x, weight, bias):
    B, C, H, W = x.shape
    xt = x.transpose(0, 2, 1, 3)
    w2 = _fold_weights(weight, bias)
    return _build_call(B, H, W)(xt, w2, _hbands())


# speedup vs baseline: 1.1667x; 1.1667x over previous
import jax
import jax.numpy as jnp
import numpy as np
from jax.experimental import pallas as pl
from jax.experimental.pallas import tpu as pltpu

_PAD = 3
_MR = 11
_MC = 21
_TH = 32
_C = 16
_O = 64
_W = 192
_NSEG = 49
_KDIM = 800


def _tables():
    basis = []
    bidx = {}

    def bid(ar, lr, ac, lc):
        key = (ar, lr, ac, lc)
        if key not in bidx:
            bidx[key] = len(basis)
            basis.append(key)
        return bidx[key]

    T = np.zeros((_NSEG, _NSEG), np.float32)
    p = 0
    for i in (-1, 0, 1):
        for j in (-1, 0, 1):
            T[p, bid(i, 1, j, 1)] = 1.0
            p += 1
    for i in range(-2, 3):
        for j in range(-2, 3):
            if abs(i) != 2 and abs(j) != 2:
                continue
            if abs(i) == 2 and abs(j) <= 1:
                T[p, bid(-2, 5, j, 1)] = 1 / 5
            elif abs(j) == 2 and abs(i) <= 1:
                T[p, bid(i, 1, -2, 5)] = 1 / 5
            else:
                T[p, bid(-2, 5, j, 1)] += 1 / 9
                T[p, bid(i, 1, -1 if j == -2 else -2, 4)] += 1 / 9
            p += 1
    for i in range(-3, 4):
        for j in range(-3, 4):
            if abs(i) != 3 and abs(j) != 3:
                continue
            if abs(i) == 3 and abs(j) <= 2:
                T[p, bid(-3 if i == -3 else -11, 15, j, 1)] = 1 / 15
            elif abs(j) == 3 and abs(i) <= 2:
                T[p, bid(i, 1, -3 if j == -3 else -21, 25)] = 1 / 25
            else:
                T[p, bid(-3 if i == -3 else -11, 15, j, 1)] += 1 / 39
                T[p, bid(i, 1, -2 if j == -3 else -21, 24)] += 1 / 39
            p += 1
    assert p == _NSEG and len(basis) == _NSEG, (p, len(basis))
    return basis, T


_BASIS, _T = _tables()
_HCOMBOS = ((5, -2), (4, -1), (4, -2), (25, -3), (25, -21), (24, -2), (24, -21))


def _hbands():
    b = np.zeros((len(_HCOMBOS), 2 * _MC + _W, _W), np.float32)
    for ci, (L, ac) in enumerate(_HCOMBOS):
        for v in range(_W):
            u0 = v + _MC + ac
            b[ci, u0:u0 + L, v] = 1.0
    return jnp.asarray(b, dtype=jnp.bfloat16)


def _fov_kernel(x_ref, w_ref, hband_ref, o_ref, f_ref, p_ref):
    t = pl.program_id(1)

    @pl.when(t == 0)
    def _build_padded():
        x = x_ref[0]
        left = jnp.concatenate(
            [jnp.broadcast_to(x[:, :, 3:4], (x.shape[0], _C, _MC - 2)),
             x[:, :, 2:3], x[:, :, 1:2]], axis=2)
        right = jnp.concatenate(
            [x[:, :, -2:-1], x[:, :, -3:-2],
             jnp.broadcast_to(x[:, :, -4:-3], (x.shape[0], _C, _MC - 2))],
            axis=2)
        body = jnp.concatenate([left, x, right], axis=2)
        p_ref[_MR:_MR + 192] = body
        p_ref[0:_MR - 2] = jnp.broadcast_to(body[3:4], (_MR - 2, _C, 234))
        p_ref[_MR - 2] = body[2]
        p_ref[_MR - 1] = body[1]
        p_ref[_MR + 192] = body[190]
        p_ref[_MR + 193] = body[189]
        p_ref[_MR + 194:] = jnp.broadcast_to(body[188:189], (_MR - 2, _C, 234))

    tile = p_ref[pl.ds(t * _TH, _TH + 2 * _MR)]

    s2 = tile[:-1] + tile[1:]
    s4 = s2[:-2] + s2[2:]
    s5 = s4[:-1] + tile[4:]
    s8 = s4[:-4] + s4[4:]
    n15 = tile.shape[0] - 14
    s15 = s8[:n15] + s4[8:8 + n15] + s2[12:12 + n15] + tile[14:]

    nhr = _TH + 6
    hb = tile[_MR - 3:_MR + 3 + _TH].astype(jnp.bfloat16)
    hb2 = hb.reshape(nhr * _C, 2 * _MC + _W)
    hres = {}
    for ci, (L, ac) in enumerate(_HCOMBOS):
        r = jnp.dot(hb2, hband_ref[ci], preferred_element_type=jnp.float32)
        hres[(L, ac)] = r.reshape(nhr, _C, _W)

    for m, (ar, lr, ac, lc) in enumerate(_BASIS):
        if lc == 1:
            src = tile if lr == 1 else (s5 if lr == 5 else s15)
            seg = src[_MR + ar:_MR + ar + _TH, :, _MC + ac:_MC + ac + _W]
        else:
            seg = hres[(lc, ac)][3 + ar:3 + ar + _TH]
        f_ref[:, m * _C:(m + 1) * _C, :] = seg.astype(jnp.bfloat16)

    iota = jax.lax.broadcasted_iota(jnp.int32, (_TH, _KDIM - _NSEG * _C, _W), 1)
    f_ref[:, _NSEG * _C:, :] = jnp.where(
        iota == 0, 1.0, 0.0).astype(jnp.bfloat16)

    w = w_ref[...]
    res = [jnp.dot(w, f_ref[h], preferred_element_type=jnp.float32)
           for h in range(_TH)]
    o_ref[0] = jnp.stack(res, axis=1)


def _fold_weights(weight, bias):
    w2 = jnp.einsum('ocp,pm->omc', weight.reshape(_O, _C, _NSEG),
                    _T).reshape(_O, _NSEG * _C)
    pad = jnp.zeros((_O, _KDIM - _NSEG * _C - 1), weight.dtype)
    return jnp.concatenate([w2, bias[:, None], pad],
                           axis=1).astype(jnp.bfloat16)


def _build_call(B, H, W, interpret=False):
    nt = H // _TH
    return pl.pallas_call(
        _fov_kernel,
        out_shape=jax.ShapeDtypeStruct((B, _O, H, W), jnp.float32),
        grid=(B, nt),
        in_specs=[
            pl.BlockSpec((1, H, _C, W), lambda b, t: (b, 0, 0, 0)),
            pl.BlockSpec((_O, _KDIM), lambda b, t: (0, 0)),
            pl.BlockSpec((len(_HCOMBOS), 2 * _MC + _W, _W),
                         lambda b, t: (0, 0, 0)),
        ],
        out_specs=pl.BlockSpec((1, _O, _TH, W), lambda b, t: (b, 0, t, 0)),
        scratch_shapes=[
            pltpu.VMEM((_TH, _KDIM, _W), jnp.bfloat16),
            pltpu.VMEM((H + 2 * _MR, _C, W + 2 * _MC), jnp.float32),
        ],
        compiler_params=pltpu.CompilerParams(
            dimension_semantics=("parallel", "arbitrary"),
        ),
        name="foveated_conv",
        interpret=interpret,
    )


@jax.jit
def kernel(x, weight, bias):
    B, C, H, W = x.shape
    xt = x.transpose(0, 2, 1, 3)
    w2 = _fold_weights(weight, bias)
    return _build_call(B, H, W)(xt, w2, _hbands())
